# baseline (speedup 1.0000x reference)
import jax
import jax.numpy as jnp
from jax import lax
from jax.experimental import pallas as pl
from jax.experimental.pallas import tpu as pltpu
from jax.experimental.pallas import tpu_sc as plsc

B = 16384
D = 32
NC = 2
NS = 16
NW = NC * NS
CHUNK = B // NW
QROWS = CHUNK // 128
NITEMS = 1_000_000


def _body(user_hbm, item_hbm, u_hbm, sig_hbm, v_hbm, ub_hbm, ib_hbm,
          out_hbm, uidx, iidx, urows, vrows, ud, vtd, ubv, ibv, sig, outv,
          sem):
  wid = lax.axis_index("s") * NC + lax.axis_index("c")
  r0 = wid * QROWS
  base = wid * CHUNK
  iota = lax.iota(jnp.int32, 16)

  pltpu.sync_copy(user_hbm.at[pl.ds(r0, QROWS)], uidx)
  pltpu.sync_copy(item_hbm.at[pl.ds(r0, QROWS)], iidx)
  pltpu.sync_copy(sig_hbm, sig)

  copies = []
  for q in range(QROWS):
    copies.append(pltpu.async_copy(ub_hbm.at[uidx.at[q]], ubv.at[q], sem))
    copies.append(pltpu.async_copy(ib_hbm.at[iidx.at[q]], ibv.at[q], sem))
    copies.append(
        pltpu.async_copy(u_hbm.at[uidx.at[q]],
                         urows.at[pl.ds(q * 128, 128)], sem))
    copies.append(
        pltpu.async_copy(v_hbm.at[iidx.at[q]],
                         vrows.at[pl.ds(q * 128, 128)], sem))
  for cp in copies:
    cp.wait()

  dvec = iota * CHUNK

  def transpose(j, _):
    for h in range(2):
      idx = dvec + (h * 16 * CHUNK + j)
      plsc.store_scatter(ud, [idx], urows[j, pl.ds(h * 16, 16)])
      plsc.store_scatter(vtd, [idx], vrows[j, pl.ds(h * 16, 16)])
    return 0

  lax.fori_loop(0, CHUNK, transpose, 0)

  def compute(jc, _):
    row = jc // 8
    col = (jc % 8) * 16
    sig_lo = sig[pl.ds(0, 16)]
    sig_hi = sig[pl.ds(16, 16)]
    acc = ubv[row, pl.ds(col, 16)] + ibv[row, pl.ds(col, 16)]
    for d in range(D):
      sig_d = sig_lo[d] if d < 16 else sig_hi[d - 16]
      vt_chunk = vtd[pl.ds(d * CHUNK + jc * 16, 16)]
      u_chunk = ud[pl.ds(d * CHUNK + jc * 16, 16)]
      acc = acc + (sig_d * vt_chunk) * u_chunk
    outv[pl.ds(jc * 16, 16)] = acc
    return 0

  lax.fori_loop(0, CHUNK // 16, compute, 0)

  pltpu.sync_copy(outv, out_hbm.at[pl.ds(base, CHUNK)])


def _tc_transpose_body(ut_ref, vt_ref, ou_ref, ov_ref):
  ou_ref[...] = ut_ref[...].T
  ov_ref[...] = vt_ref[...].T


def _tc_transpose(ut, vt):
  cols = 2048
  grid = (NITEMS + cols - 1) // cols
  return pl.pallas_call(
      _tc_transpose_body,
      grid=(grid,),
      in_specs=[
          pl.BlockSpec((D, cols), lambda i: (0, i)),
          pl.BlockSpec((D, cols), lambda i: (0, i)),
      ],
      out_specs=[
          pl.BlockSpec((cols, D), lambda i: (i, 0)),
          pl.BlockSpec((cols, D), lambda i: (i, 0)),
      ],
      out_shape=[
          jax.ShapeDtypeStruct((NITEMS, D), jnp.float32),
          jax.ShapeDtypeStruct((NITEMS, D), jnp.float32),
      ],
  )(ut, vt)


@jax.jit
def _svd_predict(user2d, item2d, U, Sigma, V, user_bias, item_bias):
  mesh = plsc.VectorSubcoreMesh(core_axis_name="c", subcore_axis_name="s",
                                num_cores=NC, num_subcores=NS)
  return pl.kernel(
      _body,
      out_type=jax.ShapeDtypeStruct((B,), jnp.float32),
      mesh=mesh,
      compiler_params=pltpu.CompilerParams(needs_layout_passes=False,
                                           use_tc_tiling_on_sc=False),
      scratch_types=[
          pltpu.VMEM((QROWS, 128), jnp.int32),
          pltpu.VMEM((QROWS, 128), jnp.int32),
          pltpu.VMEM((CHUNK, D), jnp.float32),
          pltpu.VMEM((CHUNK, D), jnp.float32),
          pltpu.VMEM((CHUNK * D,), jnp.float32),
          pltpu.VMEM((CHUNK * D,), jnp.float32),
          pltpu.VMEM((QROWS, 128), jnp.float32),
          pltpu.VMEM((QROWS, 128), jnp.float32),
          pltpu.VMEM((D,), jnp.float32),
          pltpu.VMEM((CHUNK,), jnp.float32),
          pltpu.SemaphoreType.DMA,
      ],
  )(user2d, item2d, U, Sigma, V, user_bias, item_bias)


def kernel(user, item, U, Sigma, VT, user_bias, item_bias):
  user2d = user.reshape(B // 128, 128)
  item2d = item.reshape(B // 128, 128)
  u_bm, v_bm = _tc_transpose(U.T, VT)
  return _svd_predict(user2d, item2d, u_bm, Sigma, v_bm, user_bias,
                      item_bias)


# speedup vs baseline: 1.1791x; 1.1791x over previous
import jax
import jax.numpy as jnp
from jax import lax
from jax.experimental import pallas as pl
from jax.experimental.pallas import tpu as pltpu
from jax.experimental.pallas import tpu_sc as plsc

B = 16384
D = 32
NC = 2
NS = 16
NW = NC * NS
CHUNK = B // NW
QROWS = CHUNK // 128
NITEMS = 1_000_000


def _body(user_hbm, item_hbm, u_hbm, sig_hbm, v_hbm, ub_hbm, ib_hbm,
          out_hbm, uidx, iidx, urows, vrows, ud, vtd, ubv, ibv, sig, outv,
          sem):
  wid = lax.axis_index("s") * NC + lax.axis_index("c")
  r0 = wid * QROWS
  base = wid * CHUNK
  iota = lax.iota(jnp.int32, 16)

  pltpu.sync_copy(user_hbm.at[pl.ds(r0, QROWS)], uidx)
  pltpu.sync_copy(item_hbm.at[pl.ds(r0, QROWS)], iidx)
  pltpu.sync_copy(sig_hbm, sig)

  copies = []
  for q in range(QROWS):
    copies.append(pltpu.async_copy(ub_hbm.at[uidx.at[q]], ubv.at[q], sem))
    copies.append(pltpu.async_copy(ib_hbm.at[iidx.at[q]], ibv.at[q], sem))
    copies.append(
        pltpu.async_copy(u_hbm.at[uidx.at[q]],
                         urows.at[pl.ds(q * 128, 128)], sem))
    copies.append(
        pltpu.async_copy(v_hbm.at[iidx.at[q]],
                         vrows.at[pl.ds(q * 128, 128)], sem))
  for cp in copies:
    cp.wait()

  dvec = iota * CHUNK

  def transpose(j, _):
    for h in range(2):
      idx = dvec + (h * 16 * CHUNK + j)
      plsc.store_scatter(ud, [idx], urows[j, pl.ds(h * 16, 16)])
      plsc.store_scatter(vtd, [idx], vrows[j, pl.ds(h * 16, 16)])
    return 0

  lax.fori_loop(0, CHUNK, transpose, 0)

  def compute(jc, _):
    row = jc // 8
    col = (jc % 8) * 16
    sig_lo = sig[pl.ds(0, 16)]
    sig_hi = sig[pl.ds(16, 16)]
    acc = ubv[row, pl.ds(col, 16)] + ibv[row, pl.ds(col, 16)]
    for d in range(D):
      sig_d = sig_lo[d] if d < 16 else sig_hi[d - 16]
      vt_chunk = vtd[pl.ds(d * CHUNK + jc * 16, 16)]
      u_chunk = ud[pl.ds(d * CHUNK + jc * 16, 16)]
      acc = acc + (sig_d * vt_chunk) * u_chunk
    outv[pl.ds(jc * 16, 16)] = acc
    return 0

  lax.fori_loop(0, CHUNK // 16, compute, 0)

  pltpu.sync_copy(outv, out_hbm.at[pl.ds(base, CHUNK)])


def _tc_transpose_body(ut_ref, vt_ref, ou_ref, ov_ref):
  eye = jnp.eye(D, dtype=jnp.float32)
  dims = (((0,), (0,)), ((), ()))
  ou_ref[...] = jax.lax.dot_general(ut_ref[...], eye, dims,
                                    preferred_element_type=jnp.float32)
  ov_ref[...] = jax.lax.dot_general(vt_ref[...], eye, dims,
                                    preferred_element_type=jnp.float32)


def _tc_transpose(ut, vt):
  cols = 8192
  grid = (NITEMS + cols - 1) // cols
  return pl.pallas_call(
      _tc_transpose_body,
      grid=(grid,),
      in_specs=[
          pl.BlockSpec((D, cols), lambda i: (0, i)),
          pl.BlockSpec((D, cols), lambda i: (0, i)),
      ],
      out_specs=[
          pl.BlockSpec((cols, D), lambda i: (i, 0)),
          pl.BlockSpec((cols, D), lambda i: (i, 0)),
      ],
      out_shape=[
          jax.ShapeDtypeStruct((NITEMS, D), jnp.float32),
          jax.ShapeDtypeStruct((NITEMS, D), jnp.float32),
      ],
  )(ut, vt)


@jax.jit
def _svd_predict(user2d, item2d, U, Sigma, V, user_bias, item_bias):
  mesh = plsc.VectorSubcoreMesh(core_axis_name="c", subcore_axis_name="s",
                                num_cores=NC, num_subcores=NS)
  return pl.kernel(
      _body,
      out_type=jax.ShapeDtypeStruct((B,), jnp.float32),
      mesh=mesh,
      compiler_params=pltpu.CompilerParams(needs_layout_passes=False,
                                           use_tc_tiling_on_sc=False),
      scratch_types=[
          pltpu.VMEM((QROWS, 128), jnp.int32),
          pltpu.VMEM((QROWS, 128), jnp.int32),
          pltpu.VMEM((CHUNK, D), jnp.float32),
          pltpu.VMEM((CHUNK, D), jnp.float32),
          pltpu.VMEM((CHUNK * D,), jnp.float32),
          pltpu.VMEM((CHUNK * D,), jnp.float32),
          pltpu.VMEM((QROWS, 128), jnp.float32),
          pltpu.VMEM((QROWS, 128), jnp.float32),
          pltpu.VMEM((D,), jnp.float32),
          pltpu.VMEM((CHUNK,), jnp.float32),
          pltpu.SemaphoreType.DMA,
      ],
  )(user2d, item2d, U, Sigma, V, user_bias, item_bias)


def kernel(user, item, U, Sigma, VT, user_bias, item_bias):
  user2d = user.reshape(B // 128, 128)
  item2d = item.reshape(B // 128, 128)
  u_bm, v_bm = _tc_transpose(U.T, VT)
  return _svd_predict(user2d, item2d, u_bm, Sigma, v_bm, user_bias,
                      item_bias)


# speedup vs baseline: 1.2738x; 1.0804x over previous
import jax
import jax.numpy as jnp
from jax import lax
from jax.experimental import pallas as pl
from jax.experimental.pallas import tpu as pltpu
from jax.experimental.pallas import tpu_sc as plsc

B = 16384
D = 32
NC = 2
NS = 16
NW = NC * NS
CHUNK = B // NW
QROWS = CHUNK // 128
NITEMS = 1_000_000


def _body(user_hbm, item_hbm, u_hbm, sig_hbm, v_hbm, ub_hbm, ib_hbm,
          out_hbm, uidx, iidx, urows, vrows, ud, vtd, ubv, ibv, sig, outv,
          sem):
  wid = lax.axis_index("s") * NC + lax.axis_index("c")
  r0 = wid * QROWS
  base = wid * CHUNK
  iota = lax.iota(jnp.int32, 16)

  pltpu.sync_copy(user_hbm.at[pl.ds(r0, QROWS)], uidx)
  pltpu.sync_copy(item_hbm.at[pl.ds(r0, QROWS)], iidx)
  pltpu.sync_copy(sig_hbm, sig)

  copies = []
  for q in range(QROWS):
    copies.append(pltpu.async_copy(ub_hbm.at[uidx.at[q]], ubv.at[q], sem))
    copies.append(pltpu.async_copy(ib_hbm.at[iidx.at[q]], ibv.at[q], sem))
    copies.append(
        pltpu.async_copy(u_hbm.at[uidx.at[q]],
                         urows.at[pl.ds(q * 128, 128)], sem))
    copies.append(
        pltpu.async_copy(v_hbm.at[iidx.at[q]],
                         vrows.at[pl.ds(q * 128, 128)], sem))
  for cp in copies:
    cp.wait()

  dvec = iota * CHUNK

  def transpose(j, _):
    for h in range(2):
      idx = dvec + (h * 16 * CHUNK + j)
      plsc.store_scatter(ud, [idx], urows[j, pl.ds(h * 16, 16)])
      plsc.store_scatter(vtd, [idx], vrows[j, pl.ds(h * 16, 16)])
    return 0

  lax.fori_loop(0, CHUNK, transpose, 0)

  def compute(jc, _):
    row = jc // 8
    col = (jc % 8) * 16
    sig_lo = sig[pl.ds(0, 16)]
    sig_hi = sig[pl.ds(16, 16)]
    acc = ubv[row, pl.ds(col, 16)] + ibv[row, pl.ds(col, 16)]
    for d in range(D):
      sig_d = sig_lo[d] if d < 16 else sig_hi[d - 16]
      vt_chunk = vtd[pl.ds(d * CHUNK + jc * 16, 16)]
      u_chunk = ud[pl.ds(d * CHUNK + jc * 16, 16)]
      acc = acc + (sig_d * vt_chunk) * u_chunk
    outv[pl.ds(jc * 16, 16)] = acc
    return 0

  lax.fori_loop(0, CHUNK // 16, compute, 0)

  pltpu.sync_copy(outv, out_hbm.at[pl.ds(base, CHUNK)])


def _tc_transpose_body(vt_ref, ov_ref):
  eye = jnp.eye(D, dtype=jnp.float32)
  dims = (((0,), (0,)), ((), ()))
  ov_ref[...] = jax.lax.dot_general(vt_ref[...], eye, dims,
                                    preferred_element_type=jnp.float32)


def _tc_transpose(vt):
  cols = 8192
  grid = (NITEMS + cols - 1) // cols
  return pl.pallas_call(
      _tc_transpose_body,
      grid=(grid,),
      in_specs=[pl.BlockSpec((D, cols), lambda i: (0, i))],
      out_specs=pl.BlockSpec((cols, D), lambda i: (i, 0)),
      out_shape=jax.ShapeDtypeStruct((NITEMS, D), jnp.float32),
  )(vt)


@jax.jit
def _svd_predict(user2d, item2d, U, Sigma, V, user_bias, item_bias):
  mesh = plsc.VectorSubcoreMesh(core_axis_name="c", subcore_axis_name="s",
                                num_cores=NC, num_subcores=NS)
  return pl.kernel(
      _body,
      out_type=jax.ShapeDtypeStruct((B,), jnp.float32),
      mesh=mesh,
      compiler_params=pltpu.CompilerParams(needs_layout_passes=False,
                                           use_tc_tiling_on_sc=False),
      scratch_types=[
          pltpu.VMEM((QROWS, 128), jnp.int32),
          pltpu.VMEM((QROWS, 128), jnp.int32),
          pltpu.VMEM((CHUNK, D), jnp.float32),
          pltpu.VMEM((CHUNK, D), jnp.float32),
          pltpu.VMEM((CHUNK * D,), jnp.float32),
          pltpu.VMEM((CHUNK * D,), jnp.float32),
          pltpu.VMEM((QROWS, 128), jnp.float32),
          pltpu.VMEM((QROWS, 128), jnp.float32),
          pltpu.VMEM((D,), jnp.float32),
          pltpu.VMEM((CHUNK,), jnp.float32),
          pltpu.SemaphoreType.DMA,
      ],
  )(user2d, item2d, U, Sigma, V, user_bias, item_bias)


def kernel(user, item, U, Sigma, VT, user_bias, item_bias):
  user2d = user.reshape(B // 128, 128)
  item2d = item.reshape(B // 128, 128)
  v_bm = _tc_transpose(VT)
  return _svd_predict(user2d, item2d, U, Sigma, v_bm, user_bias, item_bias)


# speedup vs baseline: 1.4425x; 1.1324x over previous
import jax
import jax.numpy as jnp
from jax import lax
from jax.experimental import pallas as pl
from jax.experimental.pallas import tpu as pltpu
from jax.experimental.pallas import tpu_sc as plsc

B = 16384
D = 32
NC = 2
NS = 16
NW = NC * NS
CHUNK = B // NW
QROWS = CHUNK // 128
NITEMS = 1_000_000


def _body(user_hbm, item_hbm, u_hbm, sig_hbm, v_hbm, ub_hbm, ib_hbm,
          out_hbm, uidx, iidx, urows, vrows, ud, vtd, ubv, ibv, sig, outv,
          sem):
  wid = lax.axis_index("s") * NC + lax.axis_index("c")
  r0 = wid * QROWS
  base = wid * CHUNK
  iota = lax.iota(jnp.int32, 16)

  pltpu.sync_copy(user_hbm.at[pl.ds(r0, QROWS)], uidx)
  pltpu.sync_copy(item_hbm.at[pl.ds(r0, QROWS)], iidx)
  pltpu.sync_copy(sig_hbm, sig)

  copies = []
  for q in range(QROWS):
    copies.append(pltpu.async_copy(ub_hbm.at[uidx.at[q]], ubv.at[q], sem))
    copies.append(pltpu.async_copy(ib_hbm.at[iidx.at[q]], ibv.at[q], sem))
    copies.append(
        pltpu.async_copy(u_hbm.at[uidx.at[q]],
                         urows.at[pl.ds(q * 128, 128)], sem))
    copies.append(
        pltpu.async_copy(v_hbm.at[iidx.at[q]],
                         vrows.at[pl.ds(q * 128, 128)], sem))
  for cp in copies:
    cp.wait()

  dvec = iota * CHUNK

  def transpose(j, _):
    for h in range(2):
      idx = dvec + (h * 16 * CHUNK + j)
      plsc.store_scatter(ud, [idx], urows[j, pl.ds(h * 16, 16)])
      plsc.store_scatter(vtd, [idx], vrows[j, pl.ds(h * 16, 16)])
    return 0

  lax.fori_loop(0, CHUNK, transpose, 0)

  def compute(jc, _):
    row = jc // 8
    col = (jc % 8) * 16
    sig_lo = sig[pl.ds(0, 16)]
    sig_hi = sig[pl.ds(16, 16)]
    acc = ubv[row, pl.ds(col, 16)] + ibv[row, pl.ds(col, 16)]
    for d in range(D):
      sig_d = sig_lo[d] if d < 16 else sig_hi[d - 16]
      vt_chunk = vtd[pl.ds(d * CHUNK + jc * 16, 16)]
      u_chunk = ud[pl.ds(d * CHUNK + jc * 16, 16)]
      acc = acc + (sig_d * vt_chunk) * u_chunk
    outv[pl.ds(jc * 16, 16)] = acc
    return 0

  lax.fori_loop(0, CHUNK // 16, compute, 0)

  pltpu.sync_copy(outv, out_hbm.at[pl.ds(base, CHUNK)])


@jax.jit
def _svd_predict(user2d, item2d, U, Sigma, V, user_bias, item_bias):
  mesh = plsc.VectorSubcoreMesh(core_axis_name="c", subcore_axis_name="s",
                                num_cores=NC, num_subcores=NS)
  return pl.kernel(
      _body,
      out_type=jax.ShapeDtypeStruct((B,), jnp.float32),
      mesh=mesh,
      compiler_params=pltpu.CompilerParams(needs_layout_passes=False,
                                           use_tc_tiling_on_sc=False),
      scratch_types=[
          pltpu.VMEM((QROWS, 128), jnp.int32),
          pltpu.VMEM((QROWS, 128), jnp.int32),
          pltpu.VMEM((CHUNK, D), jnp.float32),
          pltpu.VMEM((CHUNK, D), jnp.float32),
          pltpu.VMEM((CHUNK * D,), jnp.float32),
          pltpu.VMEM((CHUNK * D,), jnp.float32),
          pltpu.VMEM((QROWS, 128), jnp.float32),
          pltpu.VMEM((QROWS, 128), jnp.float32),
          pltpu.VMEM((D,), jnp.float32),
          pltpu.VMEM((CHUNK,), jnp.float32),
          pltpu.SemaphoreType.DMA,
      ],
  )(user2d, item2d, U, Sigma, V, user_bias, item_bias)


def kernel(user, item, U, Sigma, VT, user_bias, item_bias):
  user2d = user.reshape(B // 128, 128)
  item2d = item.reshape(B // 128, 128)
  return _svd_predict(user2d, item2d, U, Sigma, VT.T, user_bias, item_bias)
